# baseline (speedup 1.0000x reference)
import functools

import jax
import jax.numpy as jnp
from jax import lax
from jax.experimental import pallas as pl
from jax.experimental.pallas import tpu as pltpu
from jax.experimental.pallas import tpu_sc as plsc

SIDE = 28
N = SIDE * SIDE
CARD = 50
L = 16
NCHUNK = N // L
ROWS_MAIN = 48
GROUPS = ROWS_MAIN // L
W = 128

_INF = float("inf")


def _iota():
    return lax.iota(jnp.int32, L)


def _sort16(v):
    return jnp.sort(v)


def _permute(v, idx):
    return lax.gather(
        v, idx[:, None],
        dimension_numbers=lax.GatherDimensionNumbers(
            offset_dims=(), collapsed_slice_dims=(0,), start_index_map=(0,)),
        slice_sizes=(1,),
        mode=lax.GatherScatterMode.PROMISE_IN_BOUNDS)


def _sum_splat(v):
    it = _iota()
    for d in (8, 4, 2, 1):
        v = v + _permute(v, it ^ d)
    return v


def _merge32(s0, s1):
    rb = jnp.flip(s1)
    lo = jnp.minimum(s0, rb)
    hi = jnp.maximum(s0, rb)
    return _sort16(lo), _sort16(hi)


def _bitonic64_clean(l0, l1, l2, l3):
    m0 = jnp.minimum(l0, l2)
    m1 = jnp.minimum(l1, l3)
    M0 = jnp.maximum(l0, l2)
    M1 = jnp.maximum(l1, l3)
    u0 = jnp.minimum(m0, m1)
    u1 = jnp.maximum(m0, m1)
    v0 = jnp.minimum(M0, M1)
    v1 = jnp.maximum(M0, M1)
    return _sort16(u0), _sort16(u1), _sort16(v0), _sort16(v1)


def _merge64_keep_lo(a, b):
    lo0 = jnp.minimum(a[0], jnp.flip(b[3]))
    lo1 = jnp.minimum(a[1], jnp.flip(b[2]))
    lo2 = jnp.minimum(a[2], jnp.flip(b[1]))
    lo3 = jnp.minimum(a[3], jnp.flip(b[0]))
    return _bitonic64_clean(lo0, lo1, lo2, lo3)


def _sorted64_of_slice(x0, x1, x2, x3):
    s0, s1, s2, s3 = _sort16(x0), _sort16(x1), _sort16(x2), _sort16(x3)
    a = _merge32(s0, s1)
    b = _merge32(s2, s3)
    lo0 = jnp.minimum(a[0], jnp.flip(b[1]))
    lo1 = jnp.minimum(a[1], jnp.flip(b[0]))
    hi0 = jnp.maximum(a[0], jnp.flip(b[1]))
    hi1 = jnp.maximum(a[1], jnp.flip(b[0]))
    u0 = jnp.minimum(lo0, lo1)
    u1 = jnp.maximum(lo0, lo1)
    v0 = jnp.minimum(hi0, hi1)
    v1 = jnp.maximum(hi0, hi1)
    return _sort16(u0), _sort16(u1), _sort16(v0), _sort16(v1)


def _sc_body(p_hbm, I_hbm, J_hbm, out_hbm,
             p_v, mat_v, x48_v, x16_v, xs_v, xs2_v, asc_v, dsc_v, ml_v,
             fin_v, res_v, xy_sh, sorted_sh, t64_sh):
    c = lax.axis_index("c")
    s = lax.axis_index("s")

    pltpu.sync_copy(p_hbm, p_v)
    row_base = s * ROWS_MAIN

    pltpu.sync_copy(I_hbm.at[pl.ds(row_base, ROWS_MAIN)], mat_v)

    @pl.when(c != 0)
    def _():
        pltpu.sync_copy(J_hbm.at[pl.ds(row_base, ROWS_MAIN)], mat_v)

    def group_sums(local_row0, nrows=L):
        def chunk_body(ci, accs):
            pch = p_v[pl.ds(ci * L, L)]
            return tuple(
                accs[r] + mat_v[local_row0 + r, pl.ds(ci * L, L)] * pch
                for r in range(nrows))
        accs = lax.fori_loop(
            0, NCHUNK, chunk_body,
            tuple(jnp.zeros((L,), jnp.float32) for _ in range(nrows)))
        sums = jnp.zeros((L,), jnp.float32)
        it = _iota()
        for r in range(nrows):
            sums = jnp.where(it == r, _sum_splat(accs[r]), sums)
        return sums

    for g in range(GROUPS):
        x48_v[pl.ds(g * L, L)] = group_sums(g * L)
    pltpu.sync_copy(x48_v, xy_sh.at[s, pl.ds(0, ROWS_MAIN)])

    @pl.when(s == 0)
    def _():
        pltpu.sync_copy(I_hbm.at[pl.ds(L * ROWS_MAIN, L)],
                        mat_v.at[pl.ds(0, L)])

        @pl.when(c != 0)
        def _():
            pltpu.sync_copy(J_hbm.at[pl.ds(L * ROWS_MAIN, L)],
                            mat_v.at[pl.ds(0, L)])
        x16_v[...] = group_sums(0)
        pltpu.sync_copy(x16_v, xy_sh.at[L, pl.ds(0, L)])

    plsc.subcore_barrier()

    pltpu.sync_copy(xy_sh.at[s], xs_v)
    pltpu.sync_copy(xy_sh.at[L], xs2_v)
    inf_v = jnp.full((L,), _INF, jnp.float32)
    tail = xs2_v[pl.ds(0, L)]

    x0 = xs_v[pl.ds(0, L)]
    x1 = xs_v[pl.ds(L, L)]
    x2 = xs_v[pl.ds(2 * L, L)]
    x3 = jnp.where(s == 0, tail, inf_v)
    a_list = _sorted64_of_slice(x0, x1, x2, x3)
    for k in range(4):
        asc_v[pl.ds(k * L, L)] = a_list[k]
    pltpu.sync_copy(asc_v, sorted_sh.at[0, s])

    d3 = jnp.where(s == 0, -tail, inf_v)
    d_list = _sorted64_of_slice(-x0, -x1, -x2, d3)
    for k in range(4):
        dsc_v[pl.ds(k * L, L)] = d_list[k]
    pltpu.sync_copy(dsc_v, sorted_sh.at[1, s])

    plsc.subcore_barrier()

    @pl.when(s < 2)
    def _():
        pltpu.sync_copy(sorted_sh.at[s], ml_v)
        a = tuple(ml_v[0, pl.ds(k * L, L)] for k in range(4))

        def fold(g, a):
            b = tuple(ml_v[g, pl.ds(k * L, L)] for k in range(4))
            return _merge64_keep_lo(a, b)

        a = lax.fori_loop(1, L, fold, a)
        for k in range(4):
            asc_v[pl.ds(k * L, L)] = a[k]
        pltpu.sync_copy(asc_v, t64_sh.at[s])

    plsc.subcore_barrier()

    @pl.when(s == 0)
    def _():
        pltpu.sync_copy(t64_sh, fin_v)
        it = _iota()
        z = jnp.zeros((L,), jnp.float32)
        zi = jnp.zeros((L,), jnp.int32)
        for k in range(8):
            res_v[0, pl.ds(k * L, L)] = z
        for k in range(4):
            rank = k * L + it
            mask = rank < CARD
            av = fin_v[0, pl.ds(k * L, L)]
            dv = fin_v[1, pl.ds(k * L, L)]
            plsc.store_scatter(res_v, [zi, 2 * rank], av, mask=mask)
            plsc.store_scatter(res_v, [zi, 2 * rank + 1], -dv, mask=mask)

        pltpu.sync_copy(res_v, out_hbm.at[pl.ds(c, 1)])


@jax.jit
def _sc_call(p, I, J):
    f32 = jnp.float32
    out_sd = jax.ShapeDtypeStruct((2, W), f32)
    fn = functools.partial(
        pl.kernel,
        out_type=out_sd,
        mesh=plsc.VectorSubcoreMesh(core_axis_name="c", subcore_axis_name="s"),
        compiler_params=pltpu.CompilerParams(needs_layout_passes=False),
        scratch_types=[
            pltpu.VMEM((N,), f32),
            pltpu.VMEM((ROWS_MAIN, N), f32),
            pltpu.VMEM((ROWS_MAIN,), f32),
            pltpu.VMEM((L,), f32),
            pltpu.VMEM((W,), f32),
            pltpu.VMEM((W,), f32),
            pltpu.VMEM((W,), f32),
            pltpu.VMEM((W,), f32),
            pltpu.VMEM((L, W), f32),
            pltpu.VMEM((2, W), f32),
            pltpu.VMEM((1, W), f32),
            pltpu.VMEM_SHARED((L + 1, W), f32),
            pltpu.VMEM_SHARED((2, L, W), f32),
            pltpu.VMEM_SHARED((2, W), f32),
        ],
    )(_sc_body)
    return fn(p, I, J)


def kernel(p, I, J):
    o = _sc_call(p, I, J)
    dgm1 = o[0, : 2 * CARD].reshape(CARD, 2)
    dgm2 = o[1, : 2 * CARD].reshape(CARD, 2)
    return (dgm1, dgm2)
